# baseline (speedup 1.0000x reference)
import functools
import jax
import jax.numpy as jnp
from jax.experimental import pallas as pl
from jax.experimental.pallas import tpu as pltpu

_N = 10000
_K = 32
_H = 128
_BLOCK = 80


def _body(x_ref, mm_ref, mr_ref, w_ref, b_ref, m_ref, root_ref):
    inv_k = 1.0 / _K
    acc_m = jnp.sum(mm_ref[...], axis=1) * inv_k
    acc_r = jnp.sum(mr_ref[...], axis=1) * inv_k
    z = jnp.dot(x_ref[...], w_ref[...], preferred_element_type=jnp.float32)
    m_ref[...] = jnp.maximum(z + b_ref[...] + acc_m, 0.0)
    root_ref[...] = acc_r


def kernel(x, mailbox_m, mailbox_root, W_z, b_z):
    n = x.shape[0]
    grid = (n // _BLOCK,)
    wt = W_z.T
    b2 = b_z.reshape(1, _H)
    m, root = pl.pallas_call(
        _body,
        grid=grid,
        in_specs=[
            pl.BlockSpec((_BLOCK, _H), lambda i: (i, 0)),
            pl.BlockSpec((_BLOCK, _K, _H), lambda i: (i, 0, 0)),
            pl.BlockSpec((_BLOCK, _K, _H), lambda i: (i, 0, 0)),
            pl.BlockSpec((_H, _H), lambda i: (0, 0)),
            pl.BlockSpec((1, _H), lambda i: (0, 0)),
        ],
        out_specs=[
            pl.BlockSpec((_BLOCK, _H), lambda i: (i, 0)),
            pl.BlockSpec((_BLOCK, _H), lambda i: (i, 0)),
        ],
        out_shape=[
            jax.ShapeDtypeStruct((n, _H), jnp.float32),
            jax.ShapeDtypeStruct((n, _H), jnp.float32),
        ],
        compiler_params=pltpu.CompilerParams(
            dimension_semantics=("arbitrary",),
        ),
    )(x, mailbox_m, mailbox_root, wt, b2)
    return (m, root)


# speedup vs baseline: 1.0878x; 1.0878x over previous
import functools
import jax
import jax.numpy as jnp
from jax import lax
from jax.experimental import pallas as pl
from jax.experimental.pallas import tpu as pltpu
from jax.experimental.pallas import tpu_sc as plsc

_N = 10000
_K = 32
_H = 128

_BLOCK = 200
_HEAD = 3600
_BLOCK_R = _HEAD // (_N // _BLOCK)

_NWORKERS = 32
_S = _N - _HEAD
_RPW = _S // _NWORKERS
_CH = 8
_NJ = _H // 16


def _tc_body(x_ref, mm_ref, mr_ref, w_ref, b_ref, m_ref, rh_ref):
    inv_k = 1.0 / _K
    acc_m = jnp.sum(mm_ref[...], axis=1) * inv_k
    rh_ref[...] = jnp.sum(mr_ref[...], axis=1) * inv_k
    z = jnp.dot(x_ref[...], w_ref[...], preferred_element_type=jnp.float32)
    m_ref[...] = jnp.maximum(z + b_ref[...] + acc_m, 0.0)


def _tc_m_and_head(x, mailbox_m, mailbox_root, wt, b2):
    n = x.shape[0]
    return pl.pallas_call(
        _tc_body,
        grid=(n // _BLOCK,),
        in_specs=[
            pl.BlockSpec((_BLOCK, _H), lambda i: (i, 0)),
            pl.BlockSpec((_BLOCK, _K, _H), lambda i: (i, 0, 0)),
            pl.BlockSpec((_BLOCK_R, _K, _H), lambda i: (i, 0, 0)),
            pl.BlockSpec((_H, _H), lambda i: (0, 0)),
            pl.BlockSpec((1, _H), lambda i: (0, 0)),
        ],
        out_specs=[
            pl.BlockSpec((_BLOCK, _H), lambda i: (i, 0)),
            pl.BlockSpec((_BLOCK_R, _H), lambda i: (i, 0)),
        ],
        out_shape=[
            jax.ShapeDtypeStruct((n, _H), jnp.float32),
            jax.ShapeDtypeStruct((_HEAD, _H), jnp.float32),
        ],
        compiler_params=pltpu.CompilerParams(
            dimension_semantics=("arbitrary",),
        ),
    )(x, mailbox_m, mailbox_root, wt, b2)


def _sc_root_body(mr_hbm, out_hbm, buf0, buf1, out_v, sem0, sem1):
    wid = lax.axis_index("s") * 2 + lax.axis_index("c")
    base = wid * _RPW
    nchunks = _RPW // _CH
    inv_k = 1.0 / _K

    def start(g, buf, sem):
        pltpu.async_copy(
            mr_hbm.at[pl.ds(_HEAD + base + g * _CH, _CH)], buf, sem
        )

    def wait(buf, sem):
        pltpu.make_async_copy(mr_hbm.at[pl.ds(_HEAD, _CH)], buf, sem).wait()

    def compute(buf, lg):
        def row(r, c):
            accs = tuple(buf[r, 0, pl.ds(16 * j, 16)] for j in range(_NJ))
            for k in range(1, _K):
                accs = tuple(
                    accs[j] + buf[r, k, pl.ds(16 * j, 16)] for j in range(_NJ)
                )
            for j in range(_NJ):
                out_v[lg + r, pl.ds(16 * j, 16)] = accs[j] * inv_k
            return c

        lax.fori_loop(0, _CH, row, 0)

    start(0, buf0, sem0)
    start(1, buf1, sem1)

    def pair(p, carry):
        for b, (buf, sem) in enumerate(((buf0, sem0), (buf1, sem1))):
            g = 2 * p + b

            @pl.when(g < nchunks)
            def _():
                wait(buf, sem)
                compute(buf, g * _CH)

                @pl.when(g + 2 < nchunks)
                def _():
                    start(g + 2, buf, sem)

        return carry

    lax.fori_loop(0, (nchunks + 1) // 2, pair, 0)

    pltpu.sync_copy(out_v, out_hbm.at[pl.ds(base, _RPW)])


def _sc_root_tail(mailbox_root):
    mesh = plsc.VectorSubcoreMesh(core_axis_name="c", subcore_axis_name="s")
    return pl.kernel(
        _sc_root_body,
        out_type=jax.ShapeDtypeStruct((_S, _H), jnp.float32),
        mesh=mesh,
        scratch_types=[
            pltpu.VMEM((_CH, _K, _H), jnp.float32),
            pltpu.VMEM((_CH, _K, _H), jnp.float32),
            pltpu.VMEM((_RPW, _H), jnp.float32),
            pltpu.SemaphoreType.DMA,
            pltpu.SemaphoreType.DMA,
        ],
    )(mailbox_root)


def kernel(x, mailbox_m, mailbox_root, W_z, b_z):
    wt = W_z.T
    b2 = b_z.reshape(1, _H)
    root_tail = _sc_root_tail(mailbox_root)
    m, root_head = _tc_m_and_head(x, mailbox_m, mailbox_root, wt, b2)
    root = jnp.concatenate([root_head, root_tail], axis=0)
    return (m, root)


# speedup vs baseline: 1.3688x; 1.2583x over previous
import functools
import jax
import jax.numpy as jnp
from jax.experimental import pallas as pl
from jax.experimental.pallas import tpu as pltpu

_N = 10000
_K = 32
_H = 128
_BLOCK = 200


def _body(x_ref, mm_ref, mr_ref, w_ref, b_ref, m_ref, root_ref):
    inv_k = 1.0 / _K
    acc_m = jnp.sum(mm_ref[...], axis=1) * inv_k
    acc_r = jnp.sum(mr_ref[...], axis=1) * inv_k
    z = jnp.dot(x_ref[...], w_ref[...], preferred_element_type=jnp.float32)
    m_ref[...] = jnp.maximum(z + b_ref[...] + acc_m, 0.0)
    root_ref[...] = acc_r


def kernel(x, mailbox_m, mailbox_root, W_z, b_z):
    n = x.shape[0]
    grid = (n // _BLOCK,)
    wt = W_z.T
    b2 = b_z.reshape(1, _H)
    m, root = pl.pallas_call(
        _body,
        grid=grid,
        in_specs=[
            pl.BlockSpec((_BLOCK, _H), lambda i: (i, 0)),
            pl.BlockSpec((_BLOCK, _K, _H), lambda i: (i, 0, 0)),
            pl.BlockSpec((_BLOCK, _K, _H), lambda i: (i, 0, 0)),
            pl.BlockSpec((_H, _H), lambda i: (0, 0)),
            pl.BlockSpec((1, _H), lambda i: (0, 0)),
        ],
        out_specs=[
            pl.BlockSpec((_BLOCK, _H), lambda i: (i, 0)),
            pl.BlockSpec((_BLOCK, _H), lambda i: (i, 0)),
        ],
        out_shape=[
            jax.ShapeDtypeStruct((n, _H), jnp.float32),
            jax.ShapeDtypeStruct((n, _H), jnp.float32),
        ],
        compiler_params=pltpu.CompilerParams(
            dimension_semantics=("arbitrary",),
        ),
    )(x, mailbox_m, mailbox_root, wt, b2)
    return (m, root)


# speedup vs baseline: 1.3692x; 1.0003x over previous
import functools
import jax
import jax.numpy as jnp
from jax.experimental import pallas as pl
from jax.experimental.pallas import tpu as pltpu

_N = 10000
_K = 32
_H = 128
_BLOCK = 200


def _body(x_ref, mm_ref, mr_ref, w_ref, b_ref, m_ref, root_ref):
    inv_k = 1.0 / _K
    acc_m = jnp.sum(mm_ref[...], axis=1) * inv_k
    acc_r = jnp.sum(mr_ref[...], axis=1) * inv_k
    z = jnp.dot(x_ref[...], w_ref[...], preferred_element_type=jnp.float32)
    m_ref[...] = jnp.maximum(z + b_ref[...] + acc_m, 0.0)
    root_ref[...] = acc_r


def kernel(x, mailbox_m, mailbox_root, W_z, b_z):
    n = x.shape[0]
    grid = (n // _BLOCK,)
    wt = W_z.T
    b2 = b_z.reshape(1, _H)
    m, root = pl.pallas_call(
        _body,
        grid=grid,
        in_specs=[
            pl.BlockSpec((_BLOCK, _H), lambda i: (i, 0)),
            pl.BlockSpec((_BLOCK, _K, _H), lambda i: (i, 0, 0)),
            pl.BlockSpec((_BLOCK, _K, _H), lambda i: (i, 0, 0)),
            pl.BlockSpec((_H, _H), lambda i: (0, 0)),
            pl.BlockSpec((1, _H), lambda i: (0, 0)),
        ],
        out_specs=[
            pl.BlockSpec((_BLOCK, _H), lambda i: (i, 0)),
            pl.BlockSpec((_BLOCK, _H), lambda i: (i, 0)),
        ],
        out_shape=[
            jax.ShapeDtypeStruct((n, _H), jnp.float32),
            jax.ShapeDtypeStruct((n, _H), jnp.float32),
        ],
        compiler_params=pltpu.CompilerParams(
            dimension_semantics=("parallel",),
        ),
    )(x, mailbox_m, mailbox_root, wt, b2)
    return (m, root)
